# baseline (speedup 1.0000x reference)
import functools

import jax
import jax.numpy as jnp
from jax import lax
from jax.experimental import pallas as pl
from jax.experimental.pallas import tpu as pltpu
from jax.experimental.pallas import tpu_sc as plsc

_BATCH = 16384
_DIM = 64
_NC = 2
_NS = 16
_NW = _NC * _NS
_BPW = _BATCH // _NW
_CHUNK = 128
_HALF = _BPW // 2

_PACK_W = 8192
_QB = _PACK_W // 4


def _sc_gather(uidx2d, iidx2d, user_packed, item_packed):
    mesh = plsc.VectorSubcoreMesh(core_axis_name="c", subcore_axis_name="s")

    @functools.partial(
        pl.kernel,
        out_type=[
            jax.ShapeDtypeStruct((_BATCH, 128), jnp.float32),
            jax.ShapeDtypeStruct((_BATCH, 128), jnp.float32),
        ],
        mesh=mesh,
        scratch_types=[
            pltpu.VMEM((_BPW // _CHUNK, _CHUNK), jnp.int32),
            pltpu.VMEM((_BPW // _CHUNK, _CHUNK), jnp.int32),
            pltpu.VMEM((_HALF, 128), jnp.float32),
            pltpu.VMEM((_HALF, 128), jnp.float32),
            pltpu.SemaphoreType.DMA,
        ],
    )
    def k(user_hbm, item_hbm, uidx_hbm, iidx_hbm, uout_hbm, iout_hbm,
          uidx_v, iidx_v, urows_v, irows_v, sem):
        wid = lax.axis_index("s") * _NC + lax.axis_index("c")
        base = wid * _BPW
        nidx = _BPW // _CHUNK
        pltpu.sync_copy(uidx_hbm.at[pl.ds(wid * nidx, nidx)], uidx_v)
        pltpu.sync_copy(iidx_hbm.at[pl.ds(wid * nidx, nidx)], iidx_v)
        chunks_per_half = _HALF // _CHUNK
        for h in range(2):
            copies = []
            for j in range(chunks_per_half):
                jj = h * chunks_per_half + j
                copies.append(pltpu.async_copy(
                    user_hbm.at[uidx_v.at[jj]],
                    urows_v.at[pl.ds(j * _CHUNK, _CHUNK)], sem))
                copies.append(pltpu.async_copy(
                    item_hbm.at[iidx_v.at[jj]],
                    irows_v.at[pl.ds(j * _CHUNK, _CHUNK)], sem))
            for c in copies:
                c.wait()
            pltpu.sync_copy(urows_v, uout_hbm.at[pl.ds(base + h * _HALF, _HALF)])
            pltpu.sync_copy(irows_v, iout_hbm.at[pl.ds(base + h * _HALF, _HALF)])

    return k(user_packed, item_packed, uidx2d, iidx2d)


def _pack_body(xt_ref, o_ref):
    x = xt_ref[...].astype(jnp.bfloat16)
    row = jax.lax.broadcasted_iota(jnp.int32, (_DIM, _DIM), 0)
    col = jax.lax.broadcasted_iota(jnp.int32, (_DIM, _DIM), 1)
    ident = (row == col).astype(jnp.bfloat16)
    dn = (((0,), (0,)), ((), ()))

    def t(q):
        y = jax.lax.dot_general(
            x[:, q * _QB:(q + 1) * _QB], ident, dn,
            preferred_element_type=jnp.float32)
        uy = jax.lax.bitcast_convert_type(y, jnp.uint32)
        lo = (uy[:, 0:32] + jnp.uint32(0x8000)) >> jnp.uint32(16)
        hi = (uy[:, 32:64] + jnp.uint32(0x8000)) & jnp.uint32(0xFFFF0000)
        return jax.lax.bitcast_convert_type(lo | hi, jnp.float32)

    o_ref[:, 0:32] = t(0)
    o_ref[:, 32:64] = t(1)
    o_ref[:, 64:96] = t(2)
    o_ref[:, 96:128] = t(3)


def _pack(table_t):
    n = table_t.shape[1]
    grid = (n + _PACK_W - 1) // _PACK_W
    return pl.pallas_call(
        _pack_body,
        grid=(grid,),
        in_specs=[pl.BlockSpec((_DIM, _PACK_W), lambda g: (0, g))],
        out_specs=pl.BlockSpec((_QB, 128), lambda g: (g, 0)),
        out_shape=jax.ShapeDtypeStruct((grid * _QB, 128), jnp.float32),
    )(table_t)


_TC_BLOCK = 2048


def _tc_body(u_ref, i_ref, qu_ref, qi_ref, p_ref, o_ref):
    w = p_ref[0:1, 0:_DIM]
    b = p_ref[1, 0]
    qu = qu_ref[...]
    qi = qi_ref[...]

    def unpack(g_ref, q):
        g = g_ref[...]
        h1 = jnp.where(q >= 2, g[:, 64:128], g[:, 0:64])
        h2 = jnp.where(q % 2 == 1, h1[:, 32:64], h1[:, 0:32])
        gw = jax.lax.bitcast_convert_type(h2, jnp.uint32)
        lo = jax.lax.bitcast_convert_type(gw << jnp.uint32(16), jnp.float32)
        hi = jax.lax.bitcast_convert_type(
            gw & jnp.uint32(0xFFFF0000), jnp.float32)
        return lo, hi

    ulo, uhi = unpack(u_ref, qu)
    vlo, vhi = unpack(i_ref, qi)
    wlo = p_ref[0:1, 0:32]
    whi = p_ref[0:1, 32:_DIM]
    logits = jnp.sum(ulo * vlo * wlo + uhi * vhi * whi,
                     axis=1, keepdims=True) + b
    o_ref[...] = jax.nn.sigmoid(logits)


def _tc_tail(gu, gi, qu, qi, params):
    grid = _BATCH // _TC_BLOCK
    return pl.pallas_call(
        _tc_body,
        grid=(grid,),
        in_specs=[
            pl.BlockSpec((_TC_BLOCK, 128), lambda g: (g, 0)),
            pl.BlockSpec((_TC_BLOCK, 128), lambda g: (g, 0)),
            pl.BlockSpec((_TC_BLOCK, 1), lambda g: (g, 0)),
            pl.BlockSpec((_TC_BLOCK, 1), lambda g: (g, 0)),
            pl.BlockSpec((8, 128), lambda g: (0, 0)),
        ],
        out_specs=pl.BlockSpec((_TC_BLOCK, 1), lambda g: (g, 0)),
        out_shape=jax.ShapeDtypeStruct((_BATCH, 1), jnp.float32),
    )(gu, gi, qu, qi, params)


def kernel(user_indices, item_indices, embedding_user, embedding_item,
           affine_w, affine_b):
    uidx = user_indices.astype(jnp.int32)
    iidx = item_indices.astype(jnp.int32)
    user_packed = _pack(embedding_user.T)
    item_packed = _pack(embedding_item.T)
    urow = (uidx // _PACK_W) * _QB + (uidx % _QB)
    irow = (iidx // _PACK_W) * _QB + (iidx % _QB)
    uidx2d = urow.reshape(_BATCH // _CHUNK, _CHUNK)
    iidx2d = irow.reshape(_BATCH // _CHUNK, _CHUNK)
    qu = ((uidx % _PACK_W) // _QB).reshape(_BATCH, 1)
    qi = ((iidx % _PACK_W) // _QB).reshape(_BATCH, 1)
    gu, gi = _sc_gather(uidx2d, iidx2d, user_packed, item_packed)
    params = jnp.zeros((8, 128), jnp.float32)
    params = params.at[0, 0:_DIM].set(affine_w.reshape(_DIM))
    params = params.at[1, 0].set(affine_b[0])
    return _tc_tail(gu, gi, qu, qi, params)


# speedup vs baseline: 1.3294x; 1.3294x over previous
import functools

import jax
import jax.numpy as jnp
from jax import lax
from jax.experimental import pallas as pl
from jax.experimental.pallas import tpu as pltpu
from jax.experimental.pallas import tpu_sc as plsc

_BATCH = 16384
_DIM = 64
_NC = 2
_NS = 16
_NW = _NC * _NS
_BPW = _BATCH // _NW
_CHUNK = 128
_HALF = _BPW // 2

_PACK_W = 8192
_QB = _PACK_W // 4


def _sc_gather(uidx2d, iidx2d, user_packed, item_packed):
    mesh = plsc.VectorSubcoreMesh(core_axis_name="c", subcore_axis_name="s")

    @functools.partial(
        pl.kernel,
        out_type=[
            jax.ShapeDtypeStruct((_BATCH, 128), jnp.float32),
            jax.ShapeDtypeStruct((_BATCH, 128), jnp.float32),
        ],
        mesh=mesh,
        scratch_types=[
            pltpu.VMEM((_BPW // _CHUNK, _CHUNK), jnp.int32),
            pltpu.VMEM((_BPW // _CHUNK, _CHUNK), jnp.int32),
            pltpu.VMEM((_HALF, 128), jnp.float32),
            pltpu.VMEM((_HALF, 128), jnp.float32),
            pltpu.SemaphoreType.DMA,
        ],
    )
    def k(user_hbm, item_hbm, uidx_hbm, iidx_hbm, uout_hbm, iout_hbm,
          uidx_v, iidx_v, urows_v, irows_v, sem):
        wid = lax.axis_index("s") * _NC + lax.axis_index("c")
        base = wid * _BPW
        nidx = _BPW // _CHUNK
        pltpu.sync_copy(uidx_hbm.at[pl.ds(wid * nidx, nidx)], uidx_v)
        pltpu.sync_copy(iidx_hbm.at[pl.ds(wid * nidx, nidx)], iidx_v)
        chunks_per_half = _HALF // _CHUNK
        for h in range(2):
            copies = []
            for j in range(chunks_per_half):
                jj = h * chunks_per_half + j
                copies.append(pltpu.async_copy(
                    user_hbm.at[uidx_v.at[jj]],
                    urows_v.at[pl.ds(j * _CHUNK, _CHUNK)], sem))
                copies.append(pltpu.async_copy(
                    item_hbm.at[iidx_v.at[jj]],
                    irows_v.at[pl.ds(j * _CHUNK, _CHUNK)], sem))
            for c in copies:
                c.wait()
            pltpu.sync_copy(urows_v, uout_hbm.at[pl.ds(base + h * _HALF, _HALF)])
            pltpu.sync_copy(irows_v, iout_hbm.at[pl.ds(base + h * _HALF, _HALF)])

    return k(user_packed, item_packed, uidx2d, iidx2d)


def _pack_body(xt_ref, o_ref):
    x = xt_ref[...].astype(jnp.bfloat16)
    row = jax.lax.broadcasted_iota(jnp.int32, (_DIM, _DIM), 0)
    col = jax.lax.broadcasted_iota(jnp.int32, (_DIM, _DIM), 1)
    ident = (row == col).astype(jnp.bfloat16)
    dn = (((0,), (0,)), ((), ()))

    def t(q):
        y = jax.lax.dot_general(
            x[:, q * _QB:(q + 1) * _QB], ident, dn,
            preferred_element_type=jnp.float32)
        return jax.lax.bitcast_convert_type(y, jnp.uint32)

    r = jnp.uint32(0x8000)
    m = jnp.uint32(0xFFFF0000)
    s = jnp.uint32(16)
    u0, u1, u2, u3 = t(0), t(1), t(2), t(3)
    left = ((u0 + r) >> s) | ((u2 + r) & m)
    right = ((u1 + r) >> s) | ((u3 + r) & m)
    o_ref[:, 0:_DIM] = jax.lax.bitcast_convert_type(left, jnp.float32)
    o_ref[:, _DIM:128] = jax.lax.bitcast_convert_type(right, jnp.float32)


def _pack(table_t):
    n = table_t.shape[1]
    grid = (n + _PACK_W - 1) // _PACK_W
    return pl.pallas_call(
        _pack_body,
        grid=(grid,),
        in_specs=[pl.BlockSpec((_DIM, _PACK_W), lambda g: (0, g))],
        out_specs=pl.BlockSpec((_QB, 128), lambda g: (g, 0)),
        out_shape=jax.ShapeDtypeStruct((grid * _QB, 128), jnp.float32),
    )(table_t)


_TC_BLOCK = 2048


def _tc_body(u_ref, i_ref, qu_ref, qi_ref, p_ref, o_ref):
    w = p_ref[0:1, 0:_DIM]
    b = p_ref[1, 0]
    qu = qu_ref[...]
    qi = qi_ref[...]

    def unpack(g_ref, q):
        g = g_ref[...]
        h1 = jnp.where(q % 2 == 1, g[:, _DIM:128], g[:, 0:_DIM])
        bits = jax.lax.bitcast_convert_type(h1, jnp.uint32)
        f = jnp.where(q >= 2, bits & jnp.uint32(0xFFFF0000),
                      bits << jnp.uint32(16))
        return jax.lax.bitcast_convert_type(f, jnp.float32)

    prod = unpack(u_ref, qu) * unpack(i_ref, qi)
    logits = jnp.sum(prod * w, axis=1, keepdims=True) + b
    o_ref[...] = jax.nn.sigmoid(logits)


def _tc_tail(gu, gi, qu, qi, params):
    grid = _BATCH // _TC_BLOCK
    return pl.pallas_call(
        _tc_body,
        grid=(grid,),
        in_specs=[
            pl.BlockSpec((_TC_BLOCK, 128), lambda g: (g, 0)),
            pl.BlockSpec((_TC_BLOCK, 128), lambda g: (g, 0)),
            pl.BlockSpec((_TC_BLOCK, 1), lambda g: (g, 0)),
            pl.BlockSpec((_TC_BLOCK, 1), lambda g: (g, 0)),
            pl.BlockSpec((8, 128), lambda g: (0, 0)),
        ],
        out_specs=pl.BlockSpec((_TC_BLOCK, 1), lambda g: (g, 0)),
        out_shape=jax.ShapeDtypeStruct((_BATCH, 1), jnp.float32),
    )(gu, gi, qu, qi, params)


def kernel(user_indices, item_indices, embedding_user, embedding_item,
           affine_w, affine_b):
    uidx = user_indices.astype(jnp.int32)
    iidx = item_indices.astype(jnp.int32)
    user_packed = _pack(embedding_user.T)
    item_packed = _pack(embedding_item.T)
    urow = (uidx // _PACK_W) * _QB + (uidx % _QB)
    irow = (iidx // _PACK_W) * _QB + (iidx % _QB)
    uidx2d = urow.reshape(_BATCH // _CHUNK, _CHUNK)
    iidx2d = irow.reshape(_BATCH // _CHUNK, _CHUNK)
    qu = ((uidx % _PACK_W) // _QB).reshape(_BATCH, 1)
    qi = ((iidx % _PACK_W) // _QB).reshape(_BATCH, 1)
    gu, gi = _sc_gather(uidx2d, iidx2d, user_packed, item_packed)
    params = jnp.zeros((8, 128), jnp.float32)
    params = params.at[0, 0:_DIM].set(affine_w.reshape(_DIM))
    params = params.at[1, 0].set(affine_b[0])
    return _tc_tail(gu, gi, qu, qi, params)


# speedup vs baseline: 1.5366x; 1.1558x over previous
import functools

import jax
import jax.numpy as jnp
from jax import lax
from jax.experimental import pallas as pl
from jax.experimental.pallas import tpu as pltpu
from jax.experimental.pallas import tpu_sc as plsc

_BATCH = 16384
_DIM = 64
_NC = 2
_NS = 16
_NW = _NC * _NS
_BPW = _BATCH // _NW
_CHUNK = 128
_HALF = _BPW // 2

_PACK_W = 16384
_QB = _PACK_W // 4


def _sc_gather(uidx2d, iidx2d, user_packed, item_packed):
    mesh = plsc.VectorSubcoreMesh(core_axis_name="c", subcore_axis_name="s")

    @functools.partial(
        pl.kernel,
        out_type=[
            jax.ShapeDtypeStruct((_BATCH, 128), jnp.float32),
            jax.ShapeDtypeStruct((_BATCH, 128), jnp.float32),
        ],
        mesh=mesh,
        scratch_types=[
            pltpu.VMEM((_BPW // _CHUNK, _CHUNK), jnp.int32),
            pltpu.VMEM((_BPW // _CHUNK, _CHUNK), jnp.int32),
            pltpu.VMEM((_HALF, 128), jnp.float32),
            pltpu.VMEM((_HALF, 128), jnp.float32),
            pltpu.SemaphoreType.DMA,
        ],
    )
    def k(user_hbm, item_hbm, uidx_hbm, iidx_hbm, uout_hbm, iout_hbm,
          uidx_v, iidx_v, urows_v, irows_v, sem):
        wid = lax.axis_index("s") * _NC + lax.axis_index("c")
        base = wid * _BPW
        nidx = _BPW // _CHUNK
        pltpu.sync_copy(uidx_hbm.at[pl.ds(wid * nidx, nidx)], uidx_v)
        pltpu.sync_copy(iidx_hbm.at[pl.ds(wid * nidx, nidx)], iidx_v)
        chunks_per_half = _HALF // _CHUNK
        for h in range(2):
            copies = []
            for j in range(chunks_per_half):
                jj = h * chunks_per_half + j
                copies.append(pltpu.async_copy(
                    user_hbm.at[uidx_v.at[jj]],
                    urows_v.at[pl.ds(j * _CHUNK, _CHUNK)], sem))
                copies.append(pltpu.async_copy(
                    item_hbm.at[iidx_v.at[jj]],
                    irows_v.at[pl.ds(j * _CHUNK, _CHUNK)], sem))
            for c in copies:
                c.wait()
            pltpu.sync_copy(urows_v, uout_hbm.at[pl.ds(base + h * _HALF, _HALF)])
            pltpu.sync_copy(irows_v, iout_hbm.at[pl.ds(base + h * _HALF, _HALF)])

    return k(user_packed, item_packed, uidx2d, iidx2d)


def _pack_body(xt_ref, o_ref):
    x = xt_ref[...].astype(jnp.bfloat16)
    row = jax.lax.broadcasted_iota(jnp.int32, (_DIM, _DIM), 0)
    col = jax.lax.broadcasted_iota(jnp.int32, (_DIM, _DIM), 1)
    ident = (row == col).astype(jnp.bfloat16)
    dn = (((0,), (0,)), ((), ()))

    def t(q):
        y = jax.lax.dot_general(
            x[:, q * _QB:(q + 1) * _QB], ident, dn,
            preferred_element_type=jnp.float32)
        return jax.lax.bitcast_convert_type(y, jnp.uint32)

    r = jnp.uint32(0x8000)
    m = jnp.uint32(0xFFFF0000)
    s = jnp.uint32(16)
    u0, u1, u2, u3 = t(0), t(1), t(2), t(3)
    left = ((u0 + r) >> s) | ((u2 + r) & m)
    right = ((u1 + r) >> s) | ((u3 + r) & m)
    o_ref[:, 0:_DIM] = jax.lax.bitcast_convert_type(left, jnp.float32)
    o_ref[:, _DIM:128] = jax.lax.bitcast_convert_type(right, jnp.float32)


def _pack(table_t):
    n = table_t.shape[1]
    grid = (n + _PACK_W - 1) // _PACK_W
    return pl.pallas_call(
        _pack_body,
        grid=(grid,),
        in_specs=[pl.BlockSpec((_DIM, _PACK_W), lambda g: (0, g))],
        out_specs=pl.BlockSpec((_QB, 128), lambda g: (g, 0)),
        out_shape=jax.ShapeDtypeStruct((grid * _QB, 128), jnp.float32),
    )(table_t)


_TC_BLOCK = 2048


def _tc_body(u_ref, i_ref, qu_ref, qi_ref, p_ref, o_ref):
    w = p_ref[0:1, 0:_DIM]
    b = p_ref[1, 0]
    qu = qu_ref[...]
    qi = qi_ref[...]

    def unpack(g_ref, q):
        g = g_ref[...]
        h1 = jnp.where(q % 2 == 1, g[:, _DIM:128], g[:, 0:_DIM])
        bits = jax.lax.bitcast_convert_type(h1, jnp.uint32)
        f = jnp.where(q >= 2, bits & jnp.uint32(0xFFFF0000),
                      bits << jnp.uint32(16))
        return jax.lax.bitcast_convert_type(f, jnp.float32)

    prod = unpack(u_ref, qu) * unpack(i_ref, qi)
    logits = jnp.sum(prod * w, axis=1, keepdims=True) + b
    o_ref[...] = jax.nn.sigmoid(logits)


def _tc_tail(gu, gi, qu, qi, params):
    grid = _BATCH // _TC_BLOCK
    return pl.pallas_call(
        _tc_body,
        grid=(grid,),
        in_specs=[
            pl.BlockSpec((_TC_BLOCK, 128), lambda g: (g, 0)),
            pl.BlockSpec((_TC_BLOCK, 128), lambda g: (g, 0)),
            pl.BlockSpec((_TC_BLOCK, 1), lambda g: (g, 0)),
            pl.BlockSpec((_TC_BLOCK, 1), lambda g: (g, 0)),
            pl.BlockSpec((8, 128), lambda g: (0, 0)),
        ],
        out_specs=pl.BlockSpec((_TC_BLOCK, 1), lambda g: (g, 0)),
        out_shape=jax.ShapeDtypeStruct((_BATCH, 1), jnp.float32),
    )(gu, gi, qu, qi, params)


def kernel(user_indices, item_indices, embedding_user, embedding_item,
           affine_w, affine_b):
    uidx = user_indices.astype(jnp.int32)
    iidx = item_indices.astype(jnp.int32)
    user_packed = _pack(embedding_user.T)
    item_packed = _pack(embedding_item.T)
    urow = (uidx // _PACK_W) * _QB + (uidx % _QB)
    irow = (iidx // _PACK_W) * _QB + (iidx % _QB)
    uidx2d = urow.reshape(_BATCH // _CHUNK, _CHUNK)
    iidx2d = irow.reshape(_BATCH // _CHUNK, _CHUNK)
    qu = ((uidx % _PACK_W) // _QB).reshape(_BATCH, 1)
    qi = ((iidx % _PACK_W) // _QB).reshape(_BATCH, 1)
    gu, gi = _sc_gather(uidx2d, iidx2d, user_packed, item_packed)
    params = jnp.zeros((8, 128), jnp.float32)
    params = params.at[0, 0:_DIM].set(affine_w.reshape(_DIM))
    params = params.at[1, 0].set(affine_b[0])
    return _tc_tail(gu, gi, qu, qi, params)


# speedup vs baseline: 1.6370x; 1.0654x over previous
import functools

import jax
import jax.numpy as jnp
from jax import lax
from jax.experimental import pallas as pl
from jax.experimental.pallas import tpu as pltpu
from jax.experimental.pallas import tpu_sc as plsc

_BATCH = 16384
_DIM = 64
_NC = 2
_NS = 16
_NW = _NC * _NS
_BPW = _BATCH // _NW
_CHUNK = 128
_HALF = _BPW // 2

_PACK_W = 32768
_QB = _PACK_W // 4


def _sc_gather(uidx2d, iidx2d, user_packed, item_packed):
    mesh = plsc.VectorSubcoreMesh(core_axis_name="c", subcore_axis_name="s")

    @functools.partial(
        pl.kernel,
        out_type=[
            jax.ShapeDtypeStruct((_BATCH, 128), jnp.float32),
            jax.ShapeDtypeStruct((_BATCH, 128), jnp.float32),
        ],
        mesh=mesh,
        scratch_types=[
            pltpu.VMEM((_BPW // _CHUNK, _CHUNK), jnp.int32),
            pltpu.VMEM((_BPW // _CHUNK, _CHUNK), jnp.int32),
            pltpu.VMEM((_HALF, 128), jnp.float32),
            pltpu.VMEM((_HALF, 128), jnp.float32),
            pltpu.SemaphoreType.DMA,
        ],
    )
    def k(user_hbm, item_hbm, uidx_hbm, iidx_hbm, uout_hbm, iout_hbm,
          uidx_v, iidx_v, urows_v, irows_v, sem):
        wid = lax.axis_index("s") * _NC + lax.axis_index("c")
        base = wid * _BPW
        nidx = _BPW // _CHUNK
        pltpu.sync_copy(uidx_hbm.at[pl.ds(wid * nidx, nidx)], uidx_v)
        pltpu.sync_copy(iidx_hbm.at[pl.ds(wid * nidx, nidx)], iidx_v)
        chunks_per_half = _HALF // _CHUNK
        for h in range(2):
            copies = []
            for j in range(chunks_per_half):
                jj = h * chunks_per_half + j
                copies.append(pltpu.async_copy(
                    user_hbm.at[uidx_v.at[jj]],
                    urows_v.at[pl.ds(j * _CHUNK, _CHUNK)], sem))
                copies.append(pltpu.async_copy(
                    item_hbm.at[iidx_v.at[jj]],
                    irows_v.at[pl.ds(j * _CHUNK, _CHUNK)], sem))
            for c in copies:
                c.wait()
            pltpu.sync_copy(urows_v, uout_hbm.at[pl.ds(base + h * _HALF, _HALF)])
            pltpu.sync_copy(irows_v, iout_hbm.at[pl.ds(base + h * _HALF, _HALF)])

    return k(user_packed, item_packed, uidx2d, iidx2d)


def _pack_body(xt_ref, o_ref):
    x = xt_ref[...].astype(jnp.bfloat16)
    row = jax.lax.broadcasted_iota(jnp.int32, (_DIM, _DIM), 0)
    col = jax.lax.broadcasted_iota(jnp.int32, (_DIM, _DIM), 1)
    ident = (row == col).astype(jnp.bfloat16)
    dn = (((0,), (0,)), ((), ()))

    def t(q):
        y = jax.lax.dot_general(
            x[:, q * _QB:(q + 1) * _QB], ident, dn,
            preferred_element_type=jnp.float32)
        return jax.lax.bitcast_convert_type(y, jnp.uint32)

    r = jnp.uint32(0x8000)
    m = jnp.uint32(0xFFFF0000)
    s = jnp.uint32(16)
    u0, u1, u2, u3 = t(0), t(1), t(2), t(3)
    left = ((u0 + r) >> s) | ((u2 + r) & m)
    right = ((u1 + r) >> s) | ((u3 + r) & m)
    o_ref[:, 0:_DIM] = jax.lax.bitcast_convert_type(left, jnp.float32)
    o_ref[:, _DIM:128] = jax.lax.bitcast_convert_type(right, jnp.float32)


def _pack(table_t):
    n = table_t.shape[1]
    grid = (n + _PACK_W - 1) // _PACK_W
    return pl.pallas_call(
        _pack_body,
        grid=(grid,),
        in_specs=[pl.BlockSpec((_DIM, _PACK_W), lambda g: (0, g))],
        out_specs=pl.BlockSpec((_QB, 128), lambda g: (g, 0)),
        out_shape=jax.ShapeDtypeStruct((grid * _QB, 128), jnp.float32),
    )(table_t)


_TC_BLOCK = 2048


def _tc_body(u_ref, i_ref, qu_ref, qi_ref, p_ref, o_ref):
    w = p_ref[0:1, 0:_DIM]
    b = p_ref[1, 0]
    qu = qu_ref[...]
    qi = qi_ref[...]

    def unpack(g_ref, q):
        g = g_ref[...]
        h1 = jnp.where(q % 2 == 1, g[:, _DIM:128], g[:, 0:_DIM])
        bits = jax.lax.bitcast_convert_type(h1, jnp.uint32)
        f = jnp.where(q >= 2, bits & jnp.uint32(0xFFFF0000),
                      bits << jnp.uint32(16))
        return jax.lax.bitcast_convert_type(f, jnp.float32)

    prod = unpack(u_ref, qu) * unpack(i_ref, qi)
    logits = jnp.sum(prod * w, axis=1, keepdims=True) + b
    o_ref[...] = jax.nn.sigmoid(logits)


def _tc_tail(gu, gi, qu, qi, params):
    grid = _BATCH // _TC_BLOCK
    return pl.pallas_call(
        _tc_body,
        grid=(grid,),
        in_specs=[
            pl.BlockSpec((_TC_BLOCK, 128), lambda g: (g, 0)),
            pl.BlockSpec((_TC_BLOCK, 128), lambda g: (g, 0)),
            pl.BlockSpec((_TC_BLOCK, 1), lambda g: (g, 0)),
            pl.BlockSpec((_TC_BLOCK, 1), lambda g: (g, 0)),
            pl.BlockSpec((8, 128), lambda g: (0, 0)),
        ],
        out_specs=pl.BlockSpec((_TC_BLOCK, 1), lambda g: (g, 0)),
        out_shape=jax.ShapeDtypeStruct((_BATCH, 1), jnp.float32),
    )(gu, gi, qu, qi, params)


def kernel(user_indices, item_indices, embedding_user, embedding_item,
           affine_w, affine_b):
    uidx = user_indices.astype(jnp.int32)
    iidx = item_indices.astype(jnp.int32)
    user_packed = _pack(embedding_user.T)
    item_packed = _pack(embedding_item.T)
    urow = (uidx // _PACK_W) * _QB + (uidx % _QB)
    irow = (iidx // _PACK_W) * _QB + (iidx % _QB)
    uidx2d = urow.reshape(_BATCH // _CHUNK, _CHUNK)
    iidx2d = irow.reshape(_BATCH // _CHUNK, _CHUNK)
    qu = ((uidx % _PACK_W) // _QB).reshape(_BATCH, 1)
    qi = ((iidx % _PACK_W) // _QB).reshape(_BATCH, 1)
    gu, gi = _sc_gather(uidx2d, iidx2d, user_packed, item_packed)
    params = jnp.zeros((8, 128), jnp.float32)
    params = params.at[0, 0:_DIM].set(affine_w.reshape(_DIM))
    params = params.at[1, 0].set(affine_b[0])
    return _tc_tail(gu, gi, qu, qi, params)


# speedup vs baseline: 1.6577x; 1.0126x over previous
import functools

import jax
import jax.numpy as jnp
from jax import lax
from jax.experimental import pallas as pl
from jax.experimental.pallas import tpu as pltpu
from jax.experimental.pallas import tpu_sc as plsc

_BATCH = 16384
_DIM = 64
_NC = 2
_NS = 16
_NW = _NC * _NS
_BPW = _BATCH // _NW
_CHUNK = 128
_HALF = _BPW // 2

_PACK_W = 32768
_QB = _PACK_W // 4


def _sc_gather(idx2d, packed):
    mesh = plsc.VectorSubcoreMesh(core_axis_name="c", subcore_axis_name="s")

    @functools.partial(
        pl.kernel,
        out_type=jax.ShapeDtypeStruct((_BATCH, 128), jnp.float32),
        mesh=mesh,
        scratch_types=[
            pltpu.VMEM((_BPW // _CHUNK, _CHUNK), jnp.int32),
            pltpu.VMEM((_BPW, 128), jnp.float32),
            pltpu.SemaphoreType.DMA,
        ],
    )
    def k(tab_hbm, idx_hbm, out_hbm, idx_v, rows_v, sem):
        wid = lax.axis_index("s") * _NC + lax.axis_index("c")
        base = wid * _BPW
        nidx = _BPW // _CHUNK
        pltpu.sync_copy(idx_hbm.at[pl.ds(wid * nidx, nidx)], idx_v)
        copies = []
        for j in range(nidx):
            copies.append(pltpu.async_copy(
                tab_hbm.at[idx_v.at[j]],
                rows_v.at[pl.ds(j * _CHUNK, _CHUNK)], sem))
        for c in copies:
            c.wait()
        pltpu.sync_copy(rows_v, out_hbm.at[pl.ds(base, _BPW)])

    return k(packed, idx2d)


def _pack_body(xt_ref, o_ref):
    x = xt_ref[...].astype(jnp.bfloat16)
    row = jax.lax.broadcasted_iota(jnp.int32, (_DIM, _DIM), 0)
    col = jax.lax.broadcasted_iota(jnp.int32, (_DIM, _DIM), 1)
    ident = (row == col).astype(jnp.bfloat16)
    dn = (((0,), (0,)), ((), ()))

    def t(q):
        y = jax.lax.dot_general(
            x[:, q * _QB:(q + 1) * _QB], ident, dn,
            preferred_element_type=jnp.float32)
        return jax.lax.bitcast_convert_type(y, jnp.uint32)

    r = jnp.uint32(0x8000)
    m = jnp.uint32(0xFFFF0000)
    s = jnp.uint32(16)
    u0, u1, u2, u3 = t(0), t(1), t(2), t(3)
    left = ((u0 + r) >> s) | ((u2 + r) & m)
    right = ((u1 + r) >> s) | ((u3 + r) & m)
    o_ref[:, 0:_DIM] = jax.lax.bitcast_convert_type(left, jnp.float32)
    o_ref[:, _DIM:128] = jax.lax.bitcast_convert_type(right, jnp.float32)


def _pack(table_t):
    n = table_t.shape[1]
    grid = (n + _PACK_W - 1) // _PACK_W
    return pl.pallas_call(
        _pack_body,
        grid=(grid,),
        in_specs=[pl.BlockSpec((_DIM, _PACK_W), lambda g: (0, g))],
        out_specs=pl.BlockSpec((_QB, 128), lambda g: (g, 0)),
        out_shape=jax.ShapeDtypeStruct((grid * _QB, 128), jnp.float32),
    )(table_t)


_TC_BLOCK = 2048


def _tc_body(u_ref, i_ref, qu_ref, qi_ref, p_ref, o_ref):
    w = p_ref[0:1, 0:_DIM]
    b = p_ref[1, 0]
    qu = qu_ref[...]
    qi = qi_ref[...]

    def unpack(g_ref, q):
        g = g_ref[...]
        h1 = jnp.where(q % 2 == 1, g[:, _DIM:128], g[:, 0:_DIM])
        bits = jax.lax.bitcast_convert_type(h1, jnp.uint32)
        f = jnp.where(q >= 2, bits & jnp.uint32(0xFFFF0000),
                      bits << jnp.uint32(16))
        return jax.lax.bitcast_convert_type(f, jnp.float32)

    prod = unpack(u_ref, qu) * unpack(i_ref, qi)
    logits = jnp.sum(prod * w, axis=1, keepdims=True) + b
    o_ref[...] = jax.nn.sigmoid(logits)


def _tc_tail(gu, gi, qu, qi, params):
    grid = _BATCH // _TC_BLOCK
    return pl.pallas_call(
        _tc_body,
        grid=(grid,),
        in_specs=[
            pl.BlockSpec((_TC_BLOCK, 128), lambda g: (g, 0)),
            pl.BlockSpec((_TC_BLOCK, 128), lambda g: (g, 0)),
            pl.BlockSpec((_TC_BLOCK, 1), lambda g: (g, 0)),
            pl.BlockSpec((_TC_BLOCK, 1), lambda g: (g, 0)),
            pl.BlockSpec((8, 128), lambda g: (0, 0)),
        ],
        out_specs=pl.BlockSpec((_TC_BLOCK, 1), lambda g: (g, 0)),
        out_shape=jax.ShapeDtypeStruct((_BATCH, 1), jnp.float32),
    )(gu, gi, qu, qi, params)


def kernel(user_indices, item_indices, embedding_user, embedding_item,
           affine_w, affine_b):
    uidx = user_indices.astype(jnp.int32)
    iidx = item_indices.astype(jnp.int32)
    urow = (uidx // _PACK_W) * _QB + (uidx % _QB)
    irow = (iidx // _PACK_W) * _QB + (iidx % _QB)
    uidx2d = urow.reshape(_BATCH // _CHUNK, _CHUNK)
    iidx2d = irow.reshape(_BATCH // _CHUNK, _CHUNK)
    qu = ((uidx % _PACK_W) // _QB).reshape(_BATCH, 1)
    qi = ((iidx % _PACK_W) // _QB).reshape(_BATCH, 1)
    user_packed = _pack(embedding_user.T)
    gu = _sc_gather(uidx2d, user_packed)
    item_packed = _pack(embedding_item.T)
    gi = _sc_gather(iidx2d, item_packed)
    params = jnp.zeros((8, 128), jnp.float32)
    params = params.at[0, 0:_DIM].set(affine_w.reshape(_DIM))
    params = params.at[1, 0].set(affine_b[0])
    return _tc_tail(gu, gi, qu, qi, params)
